# baseline (speedup 1.0000x reference)
import functools

import jax
import jax.numpy as jnp
from jax import lax
from jax.experimental import pallas as pl
from jax.experimental.pallas import tpu as pltpu
from jax.experimental.pallas import tpu_sc as plsc

D_MODEL = 1024
D_FF = 4096
N_EXP = 8
TOPK = 2
T = 4096
BM = 256
P = T * TOPK + N_EXP * BM
NBLK = P // BM
NC, NS, L = 2, 16, 16
NW = NC * NS
TPW = T // NW
RPW = P // NW
GC = 32


def _sc_mesh():
    return plsc.VectorSubcoreMesh(core_axis_name="c", subcore_axis_name="s",
                                  num_cores=NC, num_subcores=NS)


def _wid():
    return lax.axis_index("s") * NC + lax.axis_index("c")


def _gather_body(rows_hbm, x_hbm, xs_hbm, idx_v, buf_v, sem):
    base = _wid() * RPW
    for c in range(RPW // GC):
        pltpu.async_copy(rows_hbm.at[pl.ds(base + c * GC, GC)], idx_v,
                         sem).wait()
        pltpu.async_copy(x_hbm.at[idx_v], buf_v, sem).wait()
        pltpu.async_copy(buf_v, xs_hbm.at[pl.ds(base + c * GC, GC)],
                         sem).wait()


def _sc_gather(rows, x2d):
    f = pl.kernel(
        _gather_body,
        out_type=[jax.ShapeDtypeStruct((P, D_MODEL), jnp.float32)],
        mesh=_sc_mesh(),
        scratch_types=[
            pltpu.VMEM((GC,), jnp.int32),
            pltpu.VMEM((GC, D_MODEL), jnp.float32),
            pltpu.SemaphoreType.DMA,
        ],
    )
    return f(rows, x2d)[0]


def _combine_body(ys_hbm, pos0_hbm, pos1_hbm, out_hbm,
                  p0_v, p1_v, y0_v, y1_v, o_v, sem):
    base = _wid() * TPW
    pltpu.async_copy(pos0_hbm.at[pl.ds(base, TPW)], p0_v, sem).wait()
    pltpu.async_copy(pos1_hbm.at[pl.ds(base, TPW)], p1_v, sem).wait()
    for c in range(TPW // GC):
        pltpu.async_copy(ys_hbm.at[p0_v.at[pl.ds(c * GC, GC)]], y0_v,
                         sem).wait()
        pltpu.async_copy(ys_hbm.at[p1_v.at[pl.ds(c * GC, GC)]], y1_v,
                         sem).wait()

        def row(r, _):
            def col(cc, _2):
                o_v[r, pl.ds(cc * L, L)] = (y0_v[r, pl.ds(cc * L, L)]
                                            + y1_v[r, pl.ds(cc * L, L)])
                return 0
            lax.fori_loop(0, D_MODEL // L, col, 0)
            return 0

        lax.fori_loop(0, GC, row, 0)
        pltpu.async_copy(o_v, out_hbm.at[pl.ds(base + c * GC, GC)],
                         sem).wait()


def _sc_combine(ys, pos0, pos1):
    f = pl.kernel(
        _combine_body,
        out_type=[jax.ShapeDtypeStruct((T, D_MODEL), jnp.float32)],
        mesh=_sc_mesh(),
        scratch_types=[
            pltpu.VMEM((TPW,), jnp.int32),
            pltpu.VMEM((TPW,), jnp.int32),
            pltpu.VMEM((GC, D_MODEL), jnp.float32),
            pltpu.VMEM((GC, D_MODEL), jnp.float32),
            pltpu.VMEM((GC, D_MODEL), jnp.float32),
            pltpu.SemaphoreType.DMA,
        ],
    )
    return f(ys, pos0, pos1)[0]


def _gmm_body(be_ref, chg_ref, slot_ref, pref_ref, nxt_ref,
              xs_ref, wrow_ref, wg_hbm, wu_hbm, wd_hbm, ys_ref,
              wg_v, wu_v, wd_v, sg, su, sd):
    i = pl.program_id(0)
    s = slot_ref[i]

    def _start(e, sl):
        pltpu.make_async_copy(wg_hbm.at[e], wg_v.at[sl], sg.at[sl]).start()
        pltpu.make_async_copy(wu_hbm.at[e], wu_v.at[sl], su.at[sl]).start()
        pltpu.make_async_copy(wd_hbm.at[e], wd_v.at[sl], sd.at[sl]).start()

    def _wait(e, sl):
        pltpu.make_async_copy(wg_hbm.at[e], wg_v.at[sl], sg.at[sl]).wait()
        pltpu.make_async_copy(wu_hbm.at[e], wu_v.at[sl], su.at[sl]).wait()
        pltpu.make_async_copy(wd_hbm.at[e], wd_v.at[sl], sd.at[sl]).wait()

    @pl.when(i == 0)
    def _():
        _start(be_ref[0], 0)

    @pl.when(chg_ref[i] == 1)
    def _():
        _wait(be_ref[i], s)

    @pl.when(pref_ref[i] == 1)
    def _():
        _start(nxt_ref[i], 1 - s)

    xb = xs_ref[...].astype(jnp.bfloat16)
    wg = wg_v[s]
    wu = wu_v[s]
    wd = wd_v[s]
    g = jax.lax.dot_general(xb, wg, (((1,), (1,)), ((), ())),
                            preferred_element_type=jnp.float32)
    u = jax.lax.dot_general(xb, wu, (((1,), (1,)), ((), ())),
                            preferred_element_type=jnp.float32)
    h = (jax.nn.silu(g) * u).astype(jnp.bfloat16)
    y = jax.lax.dot_general(h, wd, (((1,), (1,)), ((), ())),
                            preferred_element_type=jnp.float32)
    wr = wrow_ref[0, 0, :]
    ys_ref[...] = y * wr[:, None]


def _gmm(xs, wrows, meta, Wg16, Wu16, Wd16):
    be, chg, slot, pref, nxt = meta
    return pl.pallas_call(
        _gmm_body,
        grid_spec=pltpu.PrefetchScalarGridSpec(
            num_scalar_prefetch=5,
            grid=(NBLK,),
            in_specs=[
                pl.BlockSpec((BM, D_MODEL), lambda i, *_: (i, 0)),
                pl.BlockSpec((1, 1, BM), lambda i, *_: (i, 0, 0)),
                pl.BlockSpec(memory_space=pl.ANY),
                pl.BlockSpec(memory_space=pl.ANY),
                pl.BlockSpec(memory_space=pl.ANY),
            ],
            out_specs=pl.BlockSpec((BM, D_MODEL), lambda i, *_: (i, 0)),
            scratch_shapes=[
                pltpu.VMEM((2, D_FF, D_MODEL), jnp.bfloat16),
                pltpu.VMEM((2, D_FF, D_MODEL), jnp.bfloat16),
                pltpu.VMEM((2, D_MODEL, D_FF), jnp.bfloat16),
                pltpu.SemaphoreType.DMA((2,)),
                pltpu.SemaphoreType.DMA((2,)),
                pltpu.SemaphoreType.DMA((2,)),
            ],
        ),
        out_shape=jax.ShapeDtypeStruct((P, D_MODEL), jnp.float32),
    )(be, chg, slot, pref, nxt, xs,
      wrows.reshape(NBLK, 1, BM), Wg16, Wu16, Wd16)


def kernel(x, Wgate, Wg, Wu, Wd):
    B, S, D = x.shape
    x2d = x.reshape(-1, D)

    gate_logits = x2d @ Wgate.T
    probs = jax.nn.softmax(gate_logits, axis=-1)
    tk_w, tk_i = jax.lax.top_k(probs, TOPK)
    tk_w = tk_w / jnp.sum(tk_w, axis=-1, keepdims=True)

    ee = tk_i.reshape(-1)
    oh = (ee[:, None] == jnp.arange(N_EXP)[None, :]).astype(jnp.int32)
    ranks = jnp.cumsum(oh, axis=0) - 1
    counts = jnp.sum(oh, axis=0)
    padded = ((counts + BM - 1) // BM) * BM
    base = jnp.concatenate([jnp.zeros((1,), jnp.int32),
                            jnp.cumsum(padded)[:-1].astype(jnp.int32)])
    rank = jnp.take_along_axis(ranks, ee[:, None], axis=1)[:, 0]
    pos = base[ee] + rank
    tok = jnp.arange(2 * T, dtype=jnp.int32) // TOPK
    rows_token = jnp.zeros((P,), jnp.int32).at[pos].set(tok)
    bounds = base + padded
    be = jnp.sum(
        (jnp.arange(NBLK)[:, None] * BM >= bounds[None, :]).astype(jnp.int32),
        axis=1).astype(jnp.int32)
    be = jnp.minimum(be, N_EXP - 1)

    diff = (be[1:] != be[:-1]).astype(jnp.int32)
    one = jnp.ones((1,), jnp.int32)
    zero = jnp.zeros((1,), jnp.int32)
    chg = jnp.concatenate([one, diff])
    slot = ((jnp.cumsum(chg) - 1) % 2).astype(jnp.int32)
    pref = jnp.concatenate([diff, zero])
    nxt = jnp.concatenate([be[1:], be[-1:]])

    wrows = jnp.zeros((P,), jnp.float32).at[pos].set(tk_w.reshape(-1))

    xs = _sc_gather(rows_token, x2d)
    ys = _gmm(xs, wrows, (be, chg, slot, pref, nxt),
              Wg.astype(jnp.bfloat16),
              Wu.astype(jnp.bfloat16),
              Wd.astype(jnp.bfloat16))
    pos2 = pos.reshape(T, TOPK)
    out = _sc_combine(ys, pos2[:, 0], pos2[:, 1])
    return out.reshape(B, S, D)


# speedup vs baseline: 1.0045x; 1.0045x over previous
import functools

import jax
import jax.numpy as jnp
from jax import lax
from jax.experimental import pallas as pl
from jax.experimental.pallas import tpu as pltpu
from jax.experimental.pallas import tpu_sc as plsc

D_MODEL = 1024
D_FF = 4096
N_EXP = 8
TOPK = 2
T = 4096
BM = 256
P = T * TOPK + N_EXP * BM
NBLK = P // BM
NC, NS, L = 2, 16, 16
NW = NC * NS
TPW = T // NW
RPW = P // NW
GC = 32


def _sc_mesh():
    return plsc.VectorSubcoreMesh(core_axis_name="c", subcore_axis_name="s",
                                  num_cores=NC, num_subcores=NS)


def _wid():
    return lax.axis_index("s") * NC + lax.axis_index("c")


def _gather_body(rows_hbm, x_hbm, xs_hbm, idx_v, buf_v, sg, sw):
    base = _wid() * RPW
    nch = RPW // GC
    pltpu.async_copy(rows_hbm.at[pl.ds(base, RPW)], idx_v, sg.at[0]).wait()
    for c in range(nch):
        b = c % 2
        if c >= 2:
            pltpu.make_async_copy(
                buf_v.at[b], xs_hbm.at[pl.ds(base + (c - 2) * GC, GC)],
                sw.at[b]).wait()
        pltpu.async_copy(x_hbm.at[idx_v.at[pl.ds(c * GC, GC)]], buf_v.at[b],
                         sg.at[b]).wait()
        pltpu.async_copy(buf_v.at[b], xs_hbm.at[pl.ds(base + c * GC, GC)],
                         sw.at[b])
    for c in range(nch - 2, nch):
        b = c % 2
        pltpu.make_async_copy(
            buf_v.at[b], xs_hbm.at[pl.ds(base + c * GC, GC)], sw.at[b]).wait()


def _sc_gather(rows, x2d):
    f = pl.kernel(
        _gather_body,
        out_type=[jax.ShapeDtypeStruct((P, D_MODEL), jnp.float32)],
        mesh=_sc_mesh(),
        scratch_types=[
            pltpu.VMEM((RPW,), jnp.int32),
            pltpu.VMEM((2, GC, D_MODEL), jnp.float32),
            pltpu.SemaphoreType.DMA((2,)),
            pltpu.SemaphoreType.DMA((2,)),
        ],
    )
    return f(rows, x2d)[0]


def _combine_body(ys_hbm, pos0_hbm, pos1_hbm, out_hbm,
                  p0_v, p1_v, y0_v, y1_v, o_v, sem, sem2):
    base = _wid() * TPW
    pltpu.async_copy(pos0_hbm.at[pl.ds(base, TPW)], p0_v, sem).wait()
    pltpu.async_copy(pos1_hbm.at[pl.ds(base, TPW)], p1_v, sem).wait()
    for c in range(TPW // GC):
        cp0 = pltpu.async_copy(ys_hbm.at[p0_v.at[pl.ds(c * GC, GC)]], y0_v,
                               sem)
        cp1 = pltpu.async_copy(ys_hbm.at[p1_v.at[pl.ds(c * GC, GC)]], y1_v,
                               sem2)
        cp0.wait()
        cp1.wait()

        def row(r, _):
            def col(cc, _2):
                o_v[r, pl.ds(cc * L, L)] = (y0_v[r, pl.ds(cc * L, L)]
                                            + y1_v[r, pl.ds(cc * L, L)])
                return 0
            lax.fori_loop(0, D_MODEL // L, col, 0)
            return 0

        lax.fori_loop(0, GC, row, 0)
        pltpu.async_copy(o_v, out_hbm.at[pl.ds(base + c * GC, GC)],
                         sem).wait()


def _sc_combine(ys, pos0, pos1):
    f = pl.kernel(
        _combine_body,
        out_type=[jax.ShapeDtypeStruct((T, D_MODEL), jnp.float32)],
        mesh=_sc_mesh(),
        scratch_types=[
            pltpu.VMEM((TPW,), jnp.int32),
            pltpu.VMEM((TPW,), jnp.int32),
            pltpu.VMEM((GC, D_MODEL), jnp.float32),
            pltpu.VMEM((GC, D_MODEL), jnp.float32),
            pltpu.VMEM((GC, D_MODEL), jnp.float32),
            pltpu.SemaphoreType.DMA,
            pltpu.SemaphoreType.DMA,
        ],
    )
    return f(ys, pos0, pos1)[0]


def _gmm_body(be_ref, chg_ref, slot_ref, pref_ref, nxt_ref,
              xs_ref, wrow_ref, wg_hbm, wu_hbm, wd_hbm, ys_ref,
              wg_v, wu_v, wd_v, sg, su, sd):
    i = pl.program_id(0)
    s = slot_ref[i]

    def _start(e, sl):
        pltpu.make_async_copy(wg_hbm.at[e], wg_v.at[sl], sg.at[sl]).start()
        pltpu.make_async_copy(wu_hbm.at[e], wu_v.at[sl], su.at[sl]).start()
        pltpu.make_async_copy(wd_hbm.at[e], wd_v.at[sl], sd.at[sl]).start()

    def _wait(e, sl):
        pltpu.make_async_copy(wg_hbm.at[e], wg_v.at[sl], sg.at[sl]).wait()
        pltpu.make_async_copy(wu_hbm.at[e], wu_v.at[sl], su.at[sl]).wait()
        pltpu.make_async_copy(wd_hbm.at[e], wd_v.at[sl], sd.at[sl]).wait()

    @pl.when(i == 0)
    def _():
        _start(be_ref[0], 0)

    @pl.when(chg_ref[i] == 1)
    def _():
        _wait(be_ref[i], s)

    @pl.when(pref_ref[i] == 1)
    def _():
        _start(nxt_ref[i], 1 - s)

    xb = xs_ref[...].astype(jnp.bfloat16)
    wg = wg_v[s]
    wu = wu_v[s]
    wd = wd_v[s]
    g = jax.lax.dot_general(xb, wg, (((1,), (1,)), ((), ())),
                            preferred_element_type=jnp.float32)
    u = jax.lax.dot_general(xb, wu, (((1,), (1,)), ((), ())),
                            preferred_element_type=jnp.float32)
    h = (jax.nn.silu(g) * u).astype(jnp.bfloat16)
    y = jax.lax.dot_general(h, wd, (((1,), (1,)), ((), ())),
                            preferred_element_type=jnp.float32)
    wr = wrow_ref[0, 0, :]
    ys_ref[...] = y * wr[:, None]


def _gmm(xs, wrows, meta, Wg16, Wu16, Wd16):
    be, chg, slot, pref, nxt = meta
    return pl.pallas_call(
        _gmm_body,
        grid_spec=pltpu.PrefetchScalarGridSpec(
            num_scalar_prefetch=5,
            grid=(NBLK,),
            in_specs=[
                pl.BlockSpec((BM, D_MODEL), lambda i, *_: (i, 0)),
                pl.BlockSpec((1, 1, BM), lambda i, *_: (i, 0, 0)),
                pl.BlockSpec(memory_space=pl.ANY),
                pl.BlockSpec(memory_space=pl.ANY),
                pl.BlockSpec(memory_space=pl.ANY),
            ],
            out_specs=pl.BlockSpec((BM, D_MODEL), lambda i, *_: (i, 0)),
            scratch_shapes=[
                pltpu.VMEM((2, D_FF, D_MODEL), jnp.bfloat16),
                pltpu.VMEM((2, D_FF, D_MODEL), jnp.bfloat16),
                pltpu.VMEM((2, D_MODEL, D_FF), jnp.bfloat16),
                pltpu.SemaphoreType.DMA((2,)),
                pltpu.SemaphoreType.DMA((2,)),
                pltpu.SemaphoreType.DMA((2,)),
            ],
        ),
        out_shape=jax.ShapeDtypeStruct((P, D_MODEL), jnp.float32),
    )(be, chg, slot, pref, nxt, xs,
      wrows.reshape(NBLK, 1, BM), Wg16, Wu16, Wd16)


def kernel(x, Wgate, Wg, Wu, Wd):
    B, S, D = x.shape
    x2d = x.reshape(-1, D)

    gate_logits = x2d @ Wgate.T
    probs = jax.nn.softmax(gate_logits, axis=-1)
    tk_w, tk_i = jax.lax.top_k(probs, TOPK)
    tk_w = tk_w / jnp.sum(tk_w, axis=-1, keepdims=True)

    ee = tk_i.reshape(-1)
    oh = (ee[:, None] == jnp.arange(N_EXP)[None, :]).astype(jnp.int32)
    ranks = jnp.cumsum(oh, axis=0) - 1
    counts = jnp.sum(oh, axis=0)
    padded = ((counts + BM - 1) // BM) * BM
    base = jnp.concatenate([jnp.zeros((1,), jnp.int32),
                            jnp.cumsum(padded)[:-1].astype(jnp.int32)])
    rank = jnp.take_along_axis(ranks, ee[:, None], axis=1)[:, 0]
    pos = base[ee] + rank
    tok = jnp.arange(2 * T, dtype=jnp.int32) // TOPK
    rows_token = jnp.zeros((P,), jnp.int32).at[pos].set(tok)
    bounds = base + padded
    be = jnp.sum(
        (jnp.arange(NBLK)[:, None] * BM >= bounds[None, :]).astype(jnp.int32),
        axis=1).astype(jnp.int32)
    be = jnp.minimum(be, N_EXP - 1)

    diff = (be[1:] != be[:-1]).astype(jnp.int32)
    one = jnp.ones((1,), jnp.int32)
    zero = jnp.zeros((1,), jnp.int32)
    chg = jnp.concatenate([one, diff])
    slot = ((jnp.cumsum(chg) - 1) % 2).astype(jnp.int32)
    pref = jnp.concatenate([diff, zero])
    nxt = jnp.concatenate([be[1:], be[-1:]])

    wrows = jnp.zeros((P,), jnp.float32).at[pos].set(tk_w.reshape(-1))

    xs = _sc_gather(rows_token, x2d)
    ys = _gmm(xs, wrows, (be, chg, slot, pref, nxt),
              Wg.astype(jnp.bfloat16),
              Wu.astype(jnp.bfloat16),
              Wd.astype(jnp.bfloat16))
    pos2 = pos.reshape(T, TOPK)
    out = _sc_combine(ys, pos2[:, 0], pos2[:, 1])
    return out.reshape(B, S, D)


# speedup vs baseline: 1.0212x; 1.0166x over previous
import functools

import jax
import jax.numpy as jnp
from jax import lax
from jax.experimental import pallas as pl
from jax.experimental.pallas import tpu as pltpu
from jax.experimental.pallas import tpu_sc as plsc

D_MODEL = 1024
D_FF = 4096
N_EXP = 8
TOPK = 2
T = 4096
BM = 256
P = T * TOPK + N_EXP * BM
NBLK = P // BM
NC, NS, L = 2, 16, 16
NW = NC * NS
TPW = T // NW
RPW = P // NW
GC = 32


def _sc_mesh():
    return plsc.VectorSubcoreMesh(core_axis_name="c", subcore_axis_name="s",
                                  num_cores=NC, num_subcores=NS)


def _wid():
    return lax.axis_index("s") * NC + lax.axis_index("c")


def _gather_body(rows_hbm, x_hbm, xs_hbm, idx_v, buf_v, sg, sw):
    base = _wid() * RPW
    nch = RPW // GC
    pltpu.async_copy(rows_hbm.at[pl.ds(base, RPW)], idx_v, sg.at[0]).wait()
    for c in range(nch):
        b = c % 2
        if c >= 2:
            pltpu.make_async_copy(
                buf_v.at[b], xs_hbm.at[pl.ds(base + (c - 2) * GC, GC)],
                sw.at[b]).wait()
        pltpu.async_copy(x_hbm.at[idx_v.at[pl.ds(c * GC, GC)]], buf_v.at[b],
                         sg.at[b]).wait()
        pltpu.async_copy(buf_v.at[b], xs_hbm.at[pl.ds(base + c * GC, GC)],
                         sw.at[b])
    for c in range(nch - 2, nch):
        b = c % 2
        pltpu.make_async_copy(
            buf_v.at[b], xs_hbm.at[pl.ds(base + c * GC, GC)], sw.at[b]).wait()


def _sc_gather(rows, x2d):
    f = pl.kernel(
        _gather_body,
        out_type=[jax.ShapeDtypeStruct((P, D_MODEL), jnp.float32)],
        mesh=_sc_mesh(),
        scratch_types=[
            pltpu.VMEM((RPW,), jnp.int32),
            pltpu.VMEM((2, GC, D_MODEL), jnp.float32),
            pltpu.SemaphoreType.DMA((2,)),
            pltpu.SemaphoreType.DMA((2,)),
        ],
    )
    return f(rows, x2d)[0]


def _combine_body(ys_hbm, pos0_hbm, pos1_hbm, out_hbm,
                  p0_v, p1_v, y0_v, y1_v, o_v, sem, sem2):
    base = _wid() * TPW
    pltpu.async_copy(pos0_hbm.at[pl.ds(base, TPW)], p0_v, sem).wait()
    pltpu.async_copy(pos1_hbm.at[pl.ds(base, TPW)], p1_v, sem).wait()
    for c in range(TPW // GC):
        cp0 = pltpu.async_copy(ys_hbm.at[p0_v.at[pl.ds(c * GC, GC)]], y0_v,
                               sem)
        cp1 = pltpu.async_copy(ys_hbm.at[p1_v.at[pl.ds(c * GC, GC)]], y1_v,
                               sem2)
        cp0.wait()
        cp1.wait()

        def row(r, _):
            for cc in range(D_MODEL // L):
                o_v[r, pl.ds(cc * L, L)] = (y0_v[r, pl.ds(cc * L, L)]
                                            + y1_v[r, pl.ds(cc * L, L)])
            return 0

        lax.fori_loop(0, GC, row, 0)
        pltpu.async_copy(o_v, out_hbm.at[pl.ds(base + c * GC, GC)],
                         sem).wait()


def _sc_combine(ys, pos0, pos1):
    f = pl.kernel(
        _combine_body,
        out_type=[jax.ShapeDtypeStruct((T, D_MODEL), jnp.float32)],
        mesh=_sc_mesh(),
        scratch_types=[
            pltpu.VMEM((TPW,), jnp.int32),
            pltpu.VMEM((TPW,), jnp.int32),
            pltpu.VMEM((GC, D_MODEL), jnp.float32),
            pltpu.VMEM((GC, D_MODEL), jnp.float32),
            pltpu.VMEM((GC, D_MODEL), jnp.float32),
            pltpu.SemaphoreType.DMA,
            pltpu.SemaphoreType.DMA,
        ],
    )
    return f(ys, pos0, pos1)[0]


def _gmm_body(be_ref, chg_ref, slot_ref, pref_ref, nxt_ref,
              xs_ref, wrow_ref, wg_hbm, wu_hbm, wd_hbm, ys_ref,
              wg_v, wu_v, wd_v, sg, su, sd):
    i = pl.program_id(0)
    s = slot_ref[i]

    def _start(e, sl):
        pltpu.make_async_copy(wg_hbm.at[e], wg_v.at[sl], sg.at[sl]).start()
        pltpu.make_async_copy(wu_hbm.at[e], wu_v.at[sl], su.at[sl]).start()
        pltpu.make_async_copy(wd_hbm.at[e], wd_v.at[sl], sd.at[sl]).start()

    def _wait(e, sl):
        pltpu.make_async_copy(wg_hbm.at[e], wg_v.at[sl], sg.at[sl]).wait()
        pltpu.make_async_copy(wu_hbm.at[e], wu_v.at[sl], su.at[sl]).wait()
        pltpu.make_async_copy(wd_hbm.at[e], wd_v.at[sl], sd.at[sl]).wait()

    @pl.when(i == 0)
    def _():
        _start(be_ref[0], 0)

    @pl.when(chg_ref[i] == 1)
    def _():
        _wait(be_ref[i], s)

    @pl.when(pref_ref[i] == 1)
    def _():
        _start(nxt_ref[i], 1 - s)

    xb = xs_ref[...].astype(jnp.bfloat16)
    wg = wg_v[s]
    wu = wu_v[s]
    wd = wd_v[s]
    g = jax.lax.dot_general(xb, wg, (((1,), (1,)), ((), ())),
                            preferred_element_type=jnp.float32)
    u = jax.lax.dot_general(xb, wu, (((1,), (1,)), ((), ())),
                            preferred_element_type=jnp.float32)
    h = (jax.nn.silu(g) * u).astype(jnp.bfloat16)
    y = jax.lax.dot_general(h, wd, (((1,), (1,)), ((), ())),
                            preferred_element_type=jnp.float32)
    wr = wrow_ref[0, 0, :]
    ys_ref[...] = y * wr[:, None]


def _gmm(xs, wrows, meta, Wg16, Wu16, Wd16):
    be, chg, slot, pref, nxt = meta
    return pl.pallas_call(
        _gmm_body,
        grid_spec=pltpu.PrefetchScalarGridSpec(
            num_scalar_prefetch=5,
            grid=(NBLK,),
            in_specs=[
                pl.BlockSpec((BM, D_MODEL), lambda i, *_: (i, 0)),
                pl.BlockSpec((1, 1, BM), lambda i, *_: (i, 0, 0)),
                pl.BlockSpec(memory_space=pl.ANY),
                pl.BlockSpec(memory_space=pl.ANY),
                pl.BlockSpec(memory_space=pl.ANY),
            ],
            out_specs=pl.BlockSpec((BM, D_MODEL), lambda i, *_: (i, 0)),
            scratch_shapes=[
                pltpu.VMEM((2, D_FF, D_MODEL), jnp.bfloat16),
                pltpu.VMEM((2, D_FF, D_MODEL), jnp.bfloat16),
                pltpu.VMEM((2, D_MODEL, D_FF), jnp.bfloat16),
                pltpu.SemaphoreType.DMA((2,)),
                pltpu.SemaphoreType.DMA((2,)),
                pltpu.SemaphoreType.DMA((2,)),
            ],
        ),
        out_shape=jax.ShapeDtypeStruct((P, D_MODEL), jnp.float32),
    )(be, chg, slot, pref, nxt, xs,
      wrows.reshape(NBLK, 1, BM), Wg16, Wu16, Wd16)


def kernel(x, Wgate, Wg, Wu, Wd):
    B, S, D = x.shape
    x2d = x.reshape(-1, D)

    gate_logits = x2d @ Wgate.T
    probs = jax.nn.softmax(gate_logits, axis=-1)
    tk_w, tk_i = jax.lax.top_k(probs, TOPK)
    tk_w = tk_w / jnp.sum(tk_w, axis=-1, keepdims=True)

    ee = tk_i.reshape(-1)
    oh = (ee[:, None] == jnp.arange(N_EXP)[None, :]).astype(jnp.int32)
    ranks = jnp.cumsum(oh, axis=0) - 1
    counts = jnp.sum(oh, axis=0)
    padded = ((counts + BM - 1) // BM) * BM
    base = jnp.concatenate([jnp.zeros((1,), jnp.int32),
                            jnp.cumsum(padded)[:-1].astype(jnp.int32)])
    rank = jnp.take_along_axis(ranks, ee[:, None], axis=1)[:, 0]
    pos = base[ee] + rank
    tok = jnp.arange(2 * T, dtype=jnp.int32) // TOPK
    rows_token = jnp.zeros((P,), jnp.int32).at[pos].set(tok)
    bounds = base + padded
    be = jnp.sum(
        (jnp.arange(NBLK)[:, None] * BM >= bounds[None, :]).astype(jnp.int32),
        axis=1).astype(jnp.int32)
    be = jnp.minimum(be, N_EXP - 1)

    diff = (be[1:] != be[:-1]).astype(jnp.int32)
    one = jnp.ones((1,), jnp.int32)
    zero = jnp.zeros((1,), jnp.int32)
    chg = jnp.concatenate([one, diff])
    slot = ((jnp.cumsum(chg) - 1) % 2).astype(jnp.int32)
    pref = jnp.concatenate([diff, zero])
    nxt = jnp.concatenate([be[1:], be[-1:]])

    wrows = jnp.zeros((P,), jnp.float32).at[pos].set(tk_w.reshape(-1))

    xs = _sc_gather(rows_token, x2d)
    ys = _gmm(xs, wrows, (be, chg, slot, pref, nxt),
              Wg.astype(jnp.bfloat16),
              Wu.astype(jnp.bfloat16),
              Wd.astype(jnp.bfloat16))
    pos2 = pos.reshape(T, TOPK)
    out = _sc_combine(ys, pos2[:, 0], pos2[:, 1])
    return out.reshape(B, S, D)
